# baseline (speedup 1.0000x reference)
import functools

import jax
import jax.numpy as jnp
from jax.experimental import pallas as pl
from jax.experimental.pallas import tpu as pltpu

N = 200000
DIN = 384
H = 192
D = 48
C = 13
BETA = 0.999
EPS_BN = 1e-5

T1 = 10000
T = 10000
NT = N // T


def _p1_kernel(feat_ref, w1_ref, b1_ref, h1t_ref, st_ref):
    i = pl.program_id(0)
    h = jnp.dot(feat_ref[...], w1_ref[...],
                preferred_element_type=jnp.float32) + b1_ref[...]
    ht = h.T
    h1t_ref[0] = ht.astype(jnp.bfloat16)

    @pl.when(i == 0)
    def _():
        st_ref[...] = jnp.zeros_like(st_ref)

    st_ref[:, 0:1] += jnp.sum(ht, axis=1, keepdims=True)
    st_ref[:, 1:2] += jnp.sum(ht * ht, axis=1, keepdims=True)


def _p23_kernel(h1t_ref, st1_ref, g1_ref, be1_ref, w2_ref, b2_ref,
                g2_ref, be2_ref, ids_ref, idsr_ref, prior_ref,
                out_ref, newp_ref,
                h2t_vmem, st2_vmem, acc_vmem):
    ph = pl.program_id(0)
    j = pl.program_id(1)

    @pl.when(ph == 0)
    def _p2():
        mu = st1_ref[:, 0:1] * (1.0 / N)
        var = st1_ref[:, 1:2] * (1.0 / N) - mu * mu
        a = g1_ref[...] * jax.lax.rsqrt(var + EPS_BN)
        c = be1_ref[...] - mu * a
        x = jnp.maximum(h1t_ref[0].astype(jnp.float32) * a + c, 0.0)
        h2t = jax.lax.dot_general(
            w2_ref[...], x, dimension_numbers=(((0,), (0,)), ((), ())),
            preferred_element_type=jnp.float32) + b2_ref[...]
        h2t_vmem[j] = h2t.astype(jnp.bfloat16)

        @pl.when(j == 0)
        def _():
            st2_vmem[...] = jnp.zeros_like(st2_vmem)

        st2_vmem[:, 0:1] += jnp.sum(h2t, axis=1, keepdims=True)
        st2_vmem[:, 1:2] += jnp.sum(h2t * h2t, axis=1, keepdims=True)

    @pl.when(ph == 1)
    def _p3():
        mu = st2_vmem[:, 0:1] * (1.0 / N)
        var = st2_vmem[:, 1:2] * (1.0 / N) - mu * mu
        a = g2_ref[...] * jax.lax.rsqrt(var + EPS_BN)
        c = be2_ref[...] - mu * a
        x = jnp.maximum(h2t_vmem[j].astype(jnp.float32) * a + c, 0.0)
        ss = jnp.sum(x * x, axis=0, keepdims=True)
        f = x * jax.lax.rsqrt(jnp.maximum(ss, 1e-24))
        out_ref[...] = jnp.concatenate([f.T, ids_ref[...]], axis=1)

        onehot = (idsr_ref[0] == jax.lax.broadcasted_iota(
            jnp.int32, (C, 1), 0)).astype(jnp.float32)
        part = jax.lax.dot_general(
            onehot, f, dimension_numbers=(((1,), (1,)), ((), ())),
            preferred_element_type=jnp.float32)
        cnt = jnp.sum(onehot, axis=1, keepdims=True)

        @pl.when(j == 0)
        def _():
            acc_vmem[...] = jnp.zeros_like(acc_vmem)

        acc_vmem[0:C, 0:D] += part
        acc_vmem[0:C, D:D + 1] += cnt

        @pl.when(j == NT - 1)
        def _():
            sums = acc_vmem[0:C, 0:D]
            counts = acc_vmem[0:C, D:D + 1]
            means = sums / jnp.maximum(counts, 1.0)
            prior = prior_ref[...]
            cur = jnp.where(counts > 0, means, prior)
            newp = BETA * prior + (1.0 - BETA) * cur
            nn = jnp.sqrt(jnp.sum(newp * newp, axis=1, keepdims=True))
            newp_ref[...] = newp / jnp.maximum(nn, 1e-12)


def kernel(feat, segment_ids, W1, b1, g1, be1, W2, b2, g2, be2, prior_ema):
    b1r = b1.reshape(1, H)
    g1c = g1.reshape(H, 1)
    be1c = be1.reshape(H, 1)
    b2c = b2.reshape(D, 1)
    g2c = g2.reshape(D, 1)
    be2c = be2.reshape(D, 1)
    ids_f = segment_ids.astype(jnp.float32).reshape(N, 1)
    ids_r = segment_ids.astype(jnp.int32).reshape(NT, 1, T)

    n1 = N // T1
    h1t, st1 = pl.pallas_call(
        _p1_kernel,
        grid=(n1,),
        in_specs=[
            pl.BlockSpec((T1, DIN), lambda i: (i, 0)),
            pl.BlockSpec((DIN, H), lambda i: (0, 0)),
            pl.BlockSpec((1, H), lambda i: (0, 0)),
        ],
        out_specs=[
            pl.BlockSpec((1, H, T1), lambda i: (i, 0, 0)),
            pl.BlockSpec((H, 8), lambda i: (0, 0)),
        ],
        out_shape=[
            jax.ShapeDtypeStruct((N // T1, H, T1), jnp.bfloat16),
            jax.ShapeDtypeStruct((H, 8), jnp.float32),
        ],
    )(feat, W1, b1r)

    out, newp = pl.pallas_call(
        _p23_kernel,
        grid=(2, NT),
        in_specs=[
            pl.BlockSpec((1, H, T), lambda p, j: (jnp.where(p == 0, j, NT - 1), 0, 0)),
            pl.BlockSpec((H, 8), lambda p, j: (0, 0)),
            pl.BlockSpec((H, 1), lambda p, j: (0, 0)),
            pl.BlockSpec((H, 1), lambda p, j: (0, 0)),
            pl.BlockSpec((H, D), lambda p, j: (0, 0)),
            pl.BlockSpec((D, 1), lambda p, j: (0, 0)),
            pl.BlockSpec((D, 1), lambda p, j: (0, 0)),
            pl.BlockSpec((D, 1), lambda p, j: (0, 0)),
            pl.BlockSpec((T, 1), lambda p, j: (jnp.where(p == 0, 0, j), 0)),
            pl.BlockSpec((1, 1, T), lambda p, j: (jnp.where(p == 0, 0, j), 0, 0)),
            pl.BlockSpec((C, D), lambda p, j: (0, 0)),
        ],
        out_specs=[
            pl.BlockSpec((T, D + 1), lambda p, j: (jnp.where(p == 0, 0, j), 0)),
            pl.BlockSpec((C, D), lambda p, j: (0, 0)),
        ],
        out_shape=[
            jax.ShapeDtypeStruct((N, D + 1), jnp.float32),
            jax.ShapeDtypeStruct((C, D), jnp.float32),
        ],
        scratch_shapes=[
            pltpu.VMEM((NT, D, T), jnp.bfloat16),
            pltpu.VMEM((D, 8), jnp.float32),
            pltpu.VMEM((16, D + 1), jnp.float32),
        ],
    )(h1t, st1, g1c, be1c, W2, b2c, g2c, be2c, ids_f, ids_r, prior_ema)

    return (out, newp)


# speedup vs baseline: 1.0030x; 1.0030x over previous
import jax
import jax.numpy as jnp
from jax.experimental import pallas as pl
from jax.experimental.pallas import tpu as pltpu

N = 200000
DIN = 384
H = 192
D = 48
C = 13
BETA = 0.999
EPS_BN = 1e-5

T1 = 10000
T = 10000
NT = N // T


def _p1_kernel(feat_ref, w1_ref, b1_ref, h1t_ref, st_ref):
    i = pl.program_id(0)
    h = jnp.dot(feat_ref[...], w1_ref[...],
                preferred_element_type=jnp.float32) + b1_ref[...]
    ht = h.T
    h1t_ref[0] = ht.astype(jnp.bfloat16)

    @pl.when(i == 0)
    def _():
        st_ref[...] = jnp.zeros_like(st_ref)

    st_ref[:, 0:1] += jnp.sum(ht, axis=1, keepdims=True)
    st_ref[:, 1:2] += jnp.sum(ht * ht, axis=1, keepdims=True)


def _p23_kernel(h1t_ref, st1_ref, g1_ref, be1_ref, w2_ref, b2_ref,
                g2_ref, be2_ref, ids_ref, idsr_ref, prior_ref,
                out_ref, newp_ref,
                h2t_vmem, st2_vmem, acc_vmem):
    ph = pl.program_id(0)
    j = pl.program_id(1)

    @pl.when(ph == 0)
    def _p2():
        mu = st1_ref[:, 0:1] * (1.0 / N)
        var = st1_ref[:, 1:2] * (1.0 / N) - mu * mu
        a = g1_ref[...] * jax.lax.rsqrt(var + EPS_BN)
        c = be1_ref[...] - mu * a
        x = jnp.maximum(h1t_ref[0].astype(jnp.float32) * a + c, 0.0)
        h2t = jax.lax.dot_general(
            w2_ref[...], x, dimension_numbers=(((0,), (0,)), ((), ())),
            preferred_element_type=jnp.float32) + b2_ref[...]
        h2t_vmem[j] = h2t.astype(jnp.bfloat16)

        @pl.when(j == 0)
        def _():
            st2_vmem[...] = jnp.zeros_like(st2_vmem)

        st2_vmem[:, 0:1] += jnp.sum(h2t, axis=1, keepdims=True)
        st2_vmem[:, 1:2] += jnp.sum(h2t * h2t, axis=1, keepdims=True)

    @pl.when(ph == 1)
    def _p3():
        mu = st2_vmem[:, 0:1] * (1.0 / N)
        var = st2_vmem[:, 1:2] * (1.0 / N) - mu * mu
        a = g2_ref[...] * jax.lax.rsqrt(var + EPS_BN)
        c = be2_ref[...] - mu * a
        x = jnp.maximum(h2t_vmem[j].astype(jnp.float32) * a + c, 0.0)
        ss = jnp.sum(x * x, axis=0, keepdims=True)
        f = x * jax.lax.rsqrt(jnp.maximum(ss, 1e-24))
        out_ref[:, 0:D] = f.T
        out_ref[:, D:D + 1] = ids_ref[...]

        onehot = (idsr_ref[0] == jax.lax.broadcasted_iota(
            jnp.int32, (C, 1), 0)).astype(jnp.float32)
        fe = jnp.concatenate(
            [f, jnp.ones((1, f.shape[1]), jnp.float32)], axis=0)
        part = jax.lax.dot_general(
            onehot, fe, dimension_numbers=(((1,), (1,)), ((), ())),
            preferred_element_type=jnp.float32)

        @pl.when(j == 0)
        def _():
            acc_vmem[...] = jnp.zeros_like(acc_vmem)

        acc_vmem[0:C, :] += part

        @pl.when(j == NT - 1)
        def _():
            sums = acc_vmem[0:C, 0:D]
            counts = acc_vmem[0:C, D:D + 1]
            means = sums / jnp.maximum(counts, 1.0)
            prior = prior_ref[...]
            cur = jnp.where(counts > 0, means, prior)
            newp = BETA * prior + (1.0 - BETA) * cur
            nn = jnp.sqrt(jnp.sum(newp * newp, axis=1, keepdims=True))
            newp_ref[...] = newp / jnp.maximum(nn, 1e-12)


def kernel(feat, segment_ids, W1, b1, g1, be1, W2, b2, g2, be2, prior_ema):
    b1r = b1.reshape(1, H)
    g1c = g1.reshape(H, 1)
    be1c = be1.reshape(H, 1)
    b2c = b2.reshape(D, 1)
    g2c = g2.reshape(D, 1)
    be2c = be2.reshape(D, 1)
    ids_f = segment_ids.astype(jnp.float32).reshape(N, 1)
    ids_r = segment_ids.astype(jnp.int32).reshape(NT, 1, T)

    n1 = N // T1
    h1t, st1 = pl.pallas_call(
        _p1_kernel,
        grid=(n1,),
        in_specs=[
            pl.BlockSpec((T1, DIN), lambda i: (i, 0)),
            pl.BlockSpec((DIN, H), lambda i: (0, 0)),
            pl.BlockSpec((1, H), lambda i: (0, 0)),
        ],
        out_specs=[
            pl.BlockSpec((1, H, T1), lambda i: (i, 0, 0)),
            pl.BlockSpec((H, 8), lambda i: (0, 0)),
        ],
        out_shape=[
            jax.ShapeDtypeStruct((N // T1, H, T1), jnp.bfloat16),
            jax.ShapeDtypeStruct((H, 8), jnp.float32),
        ],
    )(feat, W1, b1r)

    out, newp = pl.pallas_call(
        _p23_kernel,
        grid=(2, NT),
        in_specs=[
            pl.BlockSpec((1, H, T), lambda p, j: (jnp.where(p == 0, j, NT - 1), 0, 0)),
            pl.BlockSpec((H, 8), lambda p, j: (0, 0)),
            pl.BlockSpec((H, 1), lambda p, j: (0, 0)),
            pl.BlockSpec((H, 1), lambda p, j: (0, 0)),
            pl.BlockSpec((H, D), lambda p, j: (0, 0)),
            pl.BlockSpec((D, 1), lambda p, j: (0, 0)),
            pl.BlockSpec((D, 1), lambda p, j: (0, 0)),
            pl.BlockSpec((D, 1), lambda p, j: (0, 0)),
            pl.BlockSpec((T, 1), lambda p, j: (jnp.where(p == 0, 0, j), 0)),
            pl.BlockSpec((1, 1, T), lambda p, j: (jnp.where(p == 0, 0, j), 0, 0)),
            pl.BlockSpec((C, D), lambda p, j: (0, 0)),
        ],
        out_specs=[
            pl.BlockSpec((T, D + 1), lambda p, j: (jnp.where(p == 0, 0, j), 0)),
            pl.BlockSpec((C, D), lambda p, j: (0, 0)),
        ],
        out_shape=[
            jax.ShapeDtypeStruct((N, D + 1), jnp.float32),
            jax.ShapeDtypeStruct((C, D), jnp.float32),
        ],
        scratch_shapes=[
            pltpu.VMEM((NT, D, T), jnp.bfloat16),
            pltpu.VMEM((D, 8), jnp.float32),
            pltpu.VMEM((16, D + 1), jnp.float32),
        ],
    )(h1t, st1, g1c, be1c, W2, b2c, g2c, be2c, ids_f, ids_r, prior_ema)

    return (out, newp)
